# baseline (speedup 1.0000x reference)
import jax
import jax.numpy as jnp
from jax.experimental import pallas as pl

N = 10000
E = 320000
F = 128
D = 1024
K = 2


def _head_kernel(sumnorm_ref, cm_ref, wo1_ref, bo1_ref, wo2_ref, bo2_ref, out_ref):
    factor = jnp.sqrt(jnp.float32(D)) / (sumnorm_ref[0, 0] / jnp.float32(N))
    emb = jnp.max(cm_ref[...], axis=0, keepdims=True) * factor
    embb = jnp.broadcast_to(emb, (8, D))
    z = jnp.dot(embb, wo1_ref[...], preferred_element_type=jnp.float32) + bo1_ref[0]
    z = jnp.where(z > 0, z, 0.01 * z)
    z2 = jnp.dot(z, wo2_ref[...], preferred_element_type=jnp.float32) + bo2_ref[0]
    col = jax.lax.broadcasted_iota(jnp.int32, z2.shape, 1)
    valid = col < 3
    zm = jnp.where(valid, z2, -1e30)
    m = jnp.max(zm, axis=1, keepdims=True)
    lse = jnp.log(jnp.sum(jnp.where(valid, jnp.exp(zm - m), 0.0), axis=1, keepdims=True)) + m
    out_ref[...] = zm - lse


def kernel(x, edge_index, W1, b1, W2, b2, Wo1, bo1, Wo2, bo2):
    src = edge_index[0]
    dst = edge_index[1]
    deg = jax.ops.segment_sum(jnp.ones((E,), dtype=jnp.float32), dst, num_segments=N)
    norm = jax.lax.rsqrt(jnp.clip(deg, 1.0, None))

    def prop(m):
        return norm[:, None] * jax.ops.segment_sum(m[src], dst, num_segments=N)

    m0 = norm[:, None] * x
    f1 = prop(m0)
    f2 = prop(norm[:, None] * f1)
    h1 = jax.nn.relu(x @ W1[:F] + f1 @ W1[F:2 * F] + f2 @ W1[2 * F:] + b1)
    mh = norm[:, None] * h1
    g1 = prop(mh)
    g2 = prop(norm[:, None] * g1)
    h2 = h1 @ W2[:D] + g1 @ W2[D:2 * D] + g2 @ W2[2 * D:] + b2

    sumnorm = jnp.sum(jnp.linalg.norm(h2, axis=1))
    cm = jnp.max(h2.reshape(N // 8, 8, D), axis=0)

    Wo2p = jnp.zeros((256, 128), jnp.float32).at[:, :3].set(Wo2)
    bo2p = jnp.zeros((1, 128), jnp.float32).at[0, :3].set(bo2)
    out = pl.pallas_call(
        _head_kernel,
        out_shape=jax.ShapeDtypeStruct((8, 128), jnp.float32),
    )(jnp.full((8, 128), sumnorm, jnp.float32), cm, Wo1, bo1.reshape(1, 256), Wo2p, bo2p)
    return out[0:1, 0:3]


# speedup vs baseline: 2.0336x; 2.0336x over previous
import functools

import jax
import jax.numpy as jnp
from jax import lax
from jax.experimental import pallas as pl
from jax.experimental.pallas import tpu as pltpu
from jax.experimental.pallas import tpu_sc as plsc

N = 10000
E = 320000
F = 128
D = 1024

NPAD = 10240
NCSC = 2
NSUB = 16
NWORK = NCSC * NSUB
GB = 128
NB = 80
EP = NWORK * NB * GB
RSUB = NPAD // NSUB
BR = 256
NGRID = NPAD // BR
NCH = D // F


NBT = 160
NB0 = 120


def _make_prop(nc):
    mesh = plsc.VectorSubcoreMesh(core_axis_name="c", subcore_axis_name="s")
    out_type = tuple(
        jax.ShapeDtypeStruct((NCSC, NPAD, F), jnp.float32) for _ in range(nc)
    )
    scratch = [
        pltpu.VMEM((GB,), jnp.int32),
        pltpu.VMEM((GB,), jnp.int32),
        pltpu.VMEM((GB,), jnp.int32),
        pltpu.VMEM((GB, F), jnp.float32),
        pltpu.VMEM((GB, F), jnp.float32),
        pltpu.VMEM_SHARED((NPAD, F), jnp.float32),
        pltpu.SemaphoreType.DMA,
        pltpu.SemaphoreType.DMA,
    ]

    def body(*refs):
        m_refs = refs[:nc]
        src_hbm, dst_hbm, zrows_hbm = refs[nc:nc + 3]
        out_refs = refs[nc + 3:nc + 3 + nc]
        (src_r0, src_r1, dst_r, rows0, rows1, agg,
         gsem0, gsem1) = refs[nc + 3 + nc:]
        cid = lax.axis_index("c")
        sid = lax.axis_index("s")
        nb_me = jnp.where(cid == 0, NB0, NBT - NB0)
        ebase = (sid * NBT + cid * NB0) * GB
        half = nb_me // 2
        for c in range(nc):
            m_hbm = m_refs[c]
            pltpu.sync_copy(zrows_hbm, agg.at[pl.ds(sid * RSUB, RSUB)])
            plsc.subcore_barrier()
            pltpu.sync_copy(src_hbm.at[pl.ds(ebase, GB)], src_r0)
            pltpu.async_copy(m_hbm.at[src_r0], rows0, gsem0)

            def round2(i, carry):
                b0 = 2 * i
                b1 = b0 + 1
                pltpu.sync_copy(src_hbm.at[pl.ds(ebase + b1 * GB, GB)], src_r1)
                pltpu.make_async_copy(m_hbm.at[src_r0], rows0, gsem0).wait()
                pltpu.async_copy(m_hbm.at[src_r1], rows1, gsem1)
                pltpu.sync_copy(dst_hbm.at[pl.ds(ebase + b0 * GB, GB)], dst_r)
                pltpu.sync_copy(rows0, agg.at[dst_r], add=True)

                @pl.when(i < half - 1)
                def _():
                    pltpu.sync_copy(src_hbm.at[pl.ds(ebase + (b0 + 2) * GB, GB)],
                                    src_r0)

                pltpu.make_async_copy(m_hbm.at[src_r1], rows1, gsem1).wait()

                @pl.when(i < half - 1)
                def _():
                    pltpu.async_copy(m_hbm.at[src_r0], rows0, gsem0)

                pltpu.sync_copy(dst_hbm.at[pl.ds(ebase + b1 * GB, GB)], dst_r)
                pltpu.sync_copy(rows1, agg.at[dst_r], add=True)
                return carry

            lax.fori_loop(0, half, round2, 0)
            plsc.subcore_barrier()
            pltpu.sync_copy(
                agg.at[pl.ds(sid * RSUB, RSUB)],
                out_refs[c].at[cid, pl.ds(sid * RSUB, RSUB)],
            )

    return pl.kernel(body, out_type=out_type, mesh=mesh, scratch_types=scratch)


def _make_deg():
    mesh = plsc.VectorSubcoreMesh(core_axis_name="c", subcore_axis_name="s")
    out_type = jax.ShapeDtypeStruct((NCSC, NPAD, F), jnp.float32)
    scratch = [
        pltpu.VMEM((GB,), jnp.int32),
        pltpu.VMEM((GB, F), jnp.float32),
        pltpu.VMEM_SHARED((NPAD, F), jnp.float32),
    ]

    def body(dst_hbm, ones_hbm, zrows_hbm, out_ref, dst_r, ones_v, agg):
        cid = lax.axis_index("c")
        sid = lax.axis_index("s")
        wid = sid * NCSC + cid
        ebase = wid * NB * GB
        pltpu.sync_copy(ones_hbm, ones_v)
        pltpu.sync_copy(zrows_hbm, agg.at[pl.ds(sid * RSUB, RSUB)])
        plsc.subcore_barrier()

        def step(b, carry):
            pltpu.sync_copy(dst_hbm.at[pl.ds(ebase + b * GB, GB)], dst_r)
            pltpu.sync_copy(ones_v, agg.at[dst_r], add=True)
            return carry

        lax.fori_loop(0, NB, step, 0)
        plsc.subcore_barrier()
        pltpu.sync_copy(
            agg.at[pl.ds(sid * RSUB, RSUB)],
            out_ref.at[cid, pl.ds(sid * RSUB, RSUB)],
        )

    return pl.kernel(body, out_type=out_type, mesh=mesh, scratch_types=scratch)


def _normprep_body(degp_ref, x_ref, normc_ref, m0_ref):
    deg = degp_ref[0, :, 0:1] + degp_ref[1, :, 0:1]
    nrm = lax.rsqrt(jnp.maximum(deg, 1.0))
    normc = jnp.broadcast_to(nrm, (BR, F))
    normc_ref[...] = normc
    m0_ref[...] = normc * x_ref[...]


def _normprep(degp, x_p):
    return pl.pallas_call(
        _normprep_body,
        grid=(NGRID,),
        in_specs=[
            pl.BlockSpec((NCSC, BR, F), lambda i: (0, i, 0)),
            pl.BlockSpec((BR, F), lambda i: (i, 0)),
        ],
        out_specs=[
            pl.BlockSpec((BR, F), lambda i: (i, 0)),
            pl.BlockSpec((BR, F), lambda i: (i, 0)),
        ],
        out_shape=[
            jax.ShapeDtypeStruct((NPAD, F), jnp.float32),
            jax.ShapeDtypeStruct((NPAD, F), jnp.float32),
        ],
    )(degp, x_p)


def _make_combine(nc):
    def body(*refs):
        p_refs = refs[:nc]
        normc_ref = refs[nc]
        f_ref = refs[nc + 1]
        m_refs = refs[nc + 2:]
        nrm = normc_ref[:, 0:1]
        for c in range(nc):
            fc = nrm * (p_refs[c][0] + p_refs[c][1])
            f_ref[:, c * F:(c + 1) * F] = fc
            m_refs[c][...] = nrm * fc

    def call(p_list, normc):
        return pl.pallas_call(
            body,
            grid=(NGRID,),
            in_specs=[pl.BlockSpec((NCSC, BR, F), lambda i: (0, i, 0))] * nc
            + [pl.BlockSpec((BR, F), lambda i: (i, 0))],
            out_specs=[pl.BlockSpec((BR, nc * F), lambda i: (i, 0))]
            + [pl.BlockSpec((BR, F), lambda i: (i, 0))] * nc,
            out_shape=[jax.ShapeDtypeStruct((NPAD, nc * F), jnp.float32)]
            + [jax.ShapeDtypeStruct((NPAD, F), jnp.float32)] * nc,
        )(*p_list, normc)

    return call


def _layer1_body(x_ref, f1_ref, q_ref, normc_ref, w1_ref, b1_ref, h1_ref, *mh_refs):
    nrm = normc_ref[:, 0:1]
    f2 = nrm * (q_ref[0] + q_ref[1])
    h = jnp.dot(x_ref[...], w1_ref[0:F], preferred_element_type=jnp.float32)
    h = h + jnp.dot(f1_ref[...], w1_ref[F:2 * F], preferred_element_type=jnp.float32)
    h = h + jnp.dot(f2, w1_ref[2 * F:3 * F], preferred_element_type=jnp.float32)
    h = jnp.maximum(h + b1_ref[0:1, :], 0.0)
    h1_ref[...] = h
    mh = nrm * h
    for c in range(NCH):
        mh_refs[c][...] = mh[:, c * F:(c + 1) * F]


def _layer1(x_p, f1, q1, normc, W1, b1r):
    return pl.pallas_call(
        _layer1_body,
        grid=(NGRID,),
        in_specs=[
            pl.BlockSpec((BR, F), lambda i: (i, 0)),
            pl.BlockSpec((BR, F), lambda i: (i, 0)),
            pl.BlockSpec((NCSC, BR, F), lambda i: (0, i, 0)),
            pl.BlockSpec((BR, F), lambda i: (i, 0)),
            pl.BlockSpec((3 * F, D), lambda i: (0, 0)),
            pl.BlockSpec((8, D), lambda i: (0, 0)),
        ],
        out_specs=[pl.BlockSpec((BR, D), lambda i: (i, 0))]
        + [pl.BlockSpec((BR, F), lambda i: (i, 0))] * NCH,
        out_shape=[jax.ShapeDtypeStruct((NPAD, D), jnp.float32)]
        + [jax.ShapeDtypeStruct((NPAD, F), jnp.float32)] * NCH,
    )(x_p, f1, q1, normc, W1, b1r)


def _layer2_body(*refs):
    h1_ref, f1p_ref = refs[0], refs[1]
    q_refs = refs[2:2 + NCH]
    normc_ref, w2_ref, b2_ref, h2_ref = refs[2 + NCH:]
    nrm = normc_ref[:, 0:1]
    acc = jnp.dot(h1_ref[...], w2_ref[0:D], preferred_element_type=jnp.float32)
    acc = acc + jnp.dot(f1p_ref[...], w2_ref[D:2 * D], preferred_element_type=jnp.float32)
    for c in range(NCH):
        f2c = nrm * (q_refs[c][0] + q_refs[c][1])
        acc = acc + jnp.dot(
            f2c, w2_ref[2 * D + c * F:2 * D + (c + 1) * F],
            preferred_element_type=jnp.float32,
        )
    h2_ref[...] = acc + b2_ref[0:1, :]


def _layer2(h1, f1p, q2_list, normc, W2, b2r):
    return pl.pallas_call(
        _layer2_body,
        grid=(NGRID,),
        in_specs=[
            pl.BlockSpec((BR, D), lambda i: (i, 0)),
            pl.BlockSpec((BR, D), lambda i: (i, 0)),
        ]
        + [pl.BlockSpec((NCSC, BR, F), lambda i: (0, i, 0))] * NCH
        + [
            pl.BlockSpec((BR, F), lambda i: (i, 0)),
            pl.BlockSpec((3 * D, D), lambda i: (0, 0)),
            pl.BlockSpec((8, D), lambda i: (0, 0)),
        ],
        out_specs=pl.BlockSpec((BR, D), lambda i: (i, 0)),
        out_shape=jax.ShapeDtypeStruct((NPAD, D), jnp.float32),
    )(h1, f1p, *q2_list, normc, W2, b2r)


def _stats_body(h2_ref, sn_ref, cm_ref):
    i = pl.program_id(0)
    row = lax.broadcasted_iota(jnp.int32, (BR, 1), 0) + i * BR
    mask = row < N
    h = h2_ref[...]
    rn = jnp.sqrt(jnp.sum(h * h, axis=1, keepdims=True))
    total = jnp.sum(jnp.where(mask, rn, 0.0))
    hm = jnp.where(mask, h, -1e30)

    @pl.when(i == 0)
    def _():
        sn_ref[...] = jnp.zeros((8, 128), jnp.float32)
        cm_ref[...] = jnp.full((8, D), -1e30, jnp.float32)

    cm = hm[0:8]
    for k in range(1, BR // 8):
        cm = jnp.maximum(cm, hm[k * 8:(k + 1) * 8])
    sn_ref[...] = sn_ref[...] + total
    cm_ref[...] = jnp.maximum(cm_ref[...], cm)


def _stats(h2):
    return pl.pallas_call(
        _stats_body,
        grid=(NGRID,),
        in_specs=[pl.BlockSpec((BR, D), lambda i: (i, 0))],
        out_specs=[
            pl.BlockSpec((8, 128), lambda i: (0, 0)),
            pl.BlockSpec((8, D), lambda i: (0, 0)),
        ],
        out_shape=[
            jax.ShapeDtypeStruct((8, 128), jnp.float32),
            jax.ShapeDtypeStruct((8, D), jnp.float32),
        ],
    )(h2)


def _head_body(sn_ref, cm_ref, wo1_ref, bo1_ref, wo2_ref, bo2_ref, out_ref):
    factor = jnp.sqrt(jnp.float32(D)) / (sn_ref[0, 0] / jnp.float32(N))
    emb = jnp.max(cm_ref[...], axis=0, keepdims=True) * factor
    embb = jnp.broadcast_to(emb, (8, D))
    z = jnp.dot(embb, wo1_ref[...], preferred_element_type=jnp.float32) + bo1_ref[0]
    z = jnp.where(z > 0, z, 0.01 * z)
    z2 = jnp.dot(z, wo2_ref[...], preferred_element_type=jnp.float32) + bo2_ref[0]
    col = lax.broadcasted_iota(jnp.int32, z2.shape, 1)
    valid = col < 3
    zm = jnp.where(valid, z2, -1e30)
    m = jnp.max(zm, axis=1, keepdims=True)
    lse = jnp.log(jnp.sum(jnp.where(valid, jnp.exp(zm - m), 0.0), axis=1,
                          keepdims=True)) + m
    out_ref[...] = zm - lse


def _head(sn, cm8, Wo1, bo1, Wo2p, bo2p):
    return pl.pallas_call(
        _head_body,
        out_shape=jax.ShapeDtypeStruct((8, 128), jnp.float32),
    )(sn, cm8, Wo1, bo1, Wo2p, bo2p)


_prop1 = _make_prop(1)
_prop8 = _make_prop(NCH)
_deg = _make_deg()
_combine1 = _make_combine(1)
_combine8 = _make_combine(NCH)


def kernel(x, edge_index, W1, b1, W2, b2, Wo1, bo1, Wo2, bo2):
    src = edge_index[0].astype(jnp.int32)
    dst = edge_index[1].astype(jnp.int32)
    pad = jnp.full((EP - E,), N, jnp.int32)
    src_p = jnp.concatenate([src, pad])
    dst_p = jnp.concatenate([dst, pad])
    x_p = jnp.pad(x, ((0, NPAD - N), (0, 0)))
    zrows = jnp.zeros((RSUB, F), jnp.float32)
    ones128 = jnp.ones((GB, F), jnp.float32)
    b1r = jnp.broadcast_to(b1[None, :], (8, D))
    b2r = jnp.broadcast_to(b2[None, :], (8, D))
    Wo2p = jnp.zeros((256, 128), jnp.float32).at[:, :3].set(Wo2)
    bo2p = jnp.zeros((1, 128), jnp.float32).at[0, :3].set(bo2)

    degp = _deg(dst_p, ones128, zrows)
    normc, m0 = _normprep(degp, x_p)

    (p1,) = _prop1(m0, src_p, dst_p, zrows)
    f1, m1 = _combine1([p1], normc)
    (q1,) = _prop1(m1, src_p, dst_p, zrows)
    h1, *mh = _layer1(x_p, f1, q1, normc, W1, b1r)

    p2 = _prop8(*mh, src_p, dst_p, zrows)
    f1p, *m2 = _combine8(list(p2), normc)
    q2 = _prop8(*m2, src_p, dst_p, zrows)
    h2 = _layer2(h1, f1p, list(q2), normc, W2, b2r)

    sn, cm8 = _stats(h2)
    out = _head(sn, cm8, Wo1, bo1.reshape(1, 256), Wo2p, bo2p)
    return out[0:1, 0:3]
